# Initial kernel scaffold; baseline (speedup 1.0000x reference)
import jax
import jax.numpy as jnp
from jax.experimental import pallas as pl


def kernel(wave, transition, target, adj_wt, adj_tt, wtp_w1, wtp_b1, wtp_w2, wtp_b2, ttp_w1, ttp_b1, ttp_w2, ttp_b2):
    raise NotImplementedError("write your pallas kernel here")


# speedup vs baseline: 11.1587x; 11.1587x over previous
import functools

import jax
import jax.numpy as jnp
from jax import lax
from jax.experimental import pallas as pl
from jax.experimental.pallas import tpu as pltpu
from jax.experimental.pallas import tpu_sc as plsc

_NW = 32
_CHUNK = 128


def _topk3_body(x_ref, vals_ref, idx_ref):
    x = x_ref[...]
    r, c = x.shape
    cols = lax.broadcasted_iota(jnp.int32, (r, c), 1)
    neg_inf = jnp.float32(-jnp.inf)
    big = jnp.int32(c)

    m1 = jnp.max(x, axis=1, keepdims=True)
    s = jnp.sum(jnp.exp(x - m1), axis=1, keepdims=True)
    a1 = jnp.min(jnp.where(x == m1, cols, big), axis=1, keepdims=True)
    x2 = jnp.where(cols == a1, neg_inf, x)
    m2 = jnp.max(x2, axis=1, keepdims=True)
    a2 = jnp.min(jnp.where(x2 == m2, cols, big), axis=1, keepdims=True)
    x3 = jnp.where(cols == a2, neg_inf, x2)
    m3 = jnp.max(x3, axis=1, keepdims=True)
    a3 = jnp.min(jnp.where(x3 == m3, cols, big), axis=1, keepdims=True)

    inv_s = 1.0 / s
    vals_ref[0, :, 0:1] = inv_s
    vals_ref[0, :, 1:2] = jnp.exp(m2 - m1) * inv_s
    vals_ref[0, :, 2:3] = jnp.exp(m3 - m1) * inv_s
    idx_ref[0, :, 0:1] = a1
    idx_ref[0, :, 1:2] = a2
    idx_ref[0, :, 2:3] = a3


def _topk3_softmax(adj, row_block):
    n, c = adj.shape
    g = n // row_block
    vals, idx = pl.pallas_call(
        _topk3_body,
        grid=(g,),
        in_specs=[pl.BlockSpec((row_block, c), lambda i: (i, 0))],
        out_specs=[
            pl.BlockSpec((1, row_block, 3), lambda i: (i, 0, 0)),
            pl.BlockSpec((1, row_block, 3), lambda i: (i, 0, 0)),
        ],
        out_shape=[
            jax.ShapeDtypeStruct((g, row_block, 3), jnp.float32),
            jax.ShapeDtypeStruct((g, row_block, 3), jnp.int32),
        ],
    )(adj)
    return vals.reshape(n, 3), idx.reshape(n, 3)


def _mm_body(x_ref, w_ref, b_ref, o_ref):
    o_ref[...] = (
        jnp.dot(x_ref[...], w_ref[...], preferred_element_type=jnp.float32)
        + b_ref[...]
    )


def _pick_row_block(n, cap=2048):
    for b in range(min(n, cap), 0, -8):
        if n % b == 0 and b % 8 == 0:
            return b
    return n


def _node_hidden(x, w, b):
    n, k = x.shape
    m = w.shape[1]
    row_block = _pick_row_block(n)
    g = n // row_block
    return pl.pallas_call(
        _mm_body,
        grid=(g,),
        in_specs=[
            pl.BlockSpec((row_block, k), lambda i: (i, 0)),
            pl.BlockSpec((k, m), lambda i: (0, 0)),
            pl.BlockSpec((1, m), lambda i: (0, 0)),
        ],
        out_specs=pl.BlockSpec((row_block, m), lambda i: (i, 0)),
        out_shape=jax.ShapeDtypeStruct((n, m), jnp.float32),
    )(x, w, b.reshape(1, m))


def _sc_gather_rows(table, idx3, n_chunks):
    v, d = table.shape
    b_pad = _NW * n_chunks * _CHUNK
    mesh = plsc.VectorSubcoreMesh(core_axis_name="c", subcore_axis_name="s")

    @functools.partial(
        pl.kernel,
        mesh=mesh,
        out_type=jax.ShapeDtypeStruct((b_pad, d), jnp.float32),
        scratch_types=[
            pltpu.VMEM((n_chunks, _CHUNK), jnp.int32),
            pltpu.VMEM((_CHUNK, d), jnp.float32),
            pltpu.SemaphoreType.DMA,
        ],
    )
    def gather_kernel(table_hbm, idx_hbm, out_hbm, idx_v, rows_v, sem):
        wid = lax.axis_index("s") * 2 + lax.axis_index("c")
        base = wid * (n_chunks * _CHUNK)
        pltpu.sync_copy(idx_hbm.at[wid], idx_v)
        for ci in range(n_chunks):
            pltpu.async_copy(table_hbm.at[idx_v.at[ci]], rows_v, sem).wait()
            pltpu.sync_copy(rows_v, out_hbm.at[pl.ds(base + ci * _CHUNK, _CHUNK)])

    return gather_kernel(table, idx3)


def _edge_weight_body(src_ref, gath_ref, vals_ref, w2_ref, b2_ref, o_ref):
    src = src_ref[...]
    g = gath_ref[...]
    h = jnp.maximum(src[:, None, :] + g, 0.0)
    z = jnp.sum(h * w2_ref[...][None, :, :], axis=-1) + b2_ref[0, 0]
    o_ref[...] = vals_ref[...] / (1.0 + jnp.exp(-z))


def _edge_weights(src_h, gath, vals, w2, b2, row_block):
    n, d = src_h.shape
    g = n // row_block
    return pl.pallas_call(
        _edge_weight_body,
        grid=(g,),
        in_specs=[
            pl.BlockSpec((row_block, d), lambda i: (i, 0)),
            pl.BlockSpec((row_block, 3, d), lambda i: (i, 0, 0)),
            pl.BlockSpec((row_block, 3), lambda i: (i, 0)),
            pl.BlockSpec((1, d), lambda i: (0, 0)),
            pl.BlockSpec((1, 1), lambda i: (0, 0)),
        ],
        out_specs=pl.BlockSpec((row_block, 3), lambda i: (i, 0)),
        out_shape=jax.ShapeDtypeStruct((n, 3), jnp.float32),
    )(src_h, gath, vals, w2.reshape(1, d), b2.reshape(1, 1))


def _edge_path(adj, src_feat, tgt_feat, w1, b1, w2, b2, adj_row_block,
               ew_row_block, n_chunks):
    n_src = adj.shape[0]
    d = src_feat.shape[1]
    vals, idx = _topk3_softmax(adj, adj_row_block)

    src_h = _node_hidden(src_feat, w1[:d], b1)
    tgt_h = _node_hidden(tgt_feat, w1[d:], jnp.zeros_like(b1))

    n_edges = n_src * 3
    b_pad = _NW * n_chunks * _CHUNK
    tgt_idx = idx.reshape(-1)
    idx_pad = jnp.zeros((b_pad,), jnp.int32).at[:n_edges].set(tgt_idx)
    gath = _sc_gather_rows(tgt_h, idx_pad.reshape(_NW, n_chunks, _CHUNK), n_chunks)
    gath = gath[:n_edges].reshape(n_src, 3, d)

    ew = _edge_weights(src_h, gath, vals, w2.reshape(-1), b2, ew_row_block)

    src_idx = jnp.repeat(jnp.arange(n_src, dtype=jnp.int32), 3)
    edge_index = jnp.stack([src_idx, tgt_idx])
    return edge_index, ew.reshape(-1)


def kernel(wave, transition, target, adj_wt, adj_tt, wtp_w1, wtp_b1, wtp_w2,
           wtp_b2, ttp_w1, ttp_b1, ttp_w2, ttp_b2):
    wt_edges, wt_weights = _edge_path(
        adj_wt, wave[0], transition[0], wtp_w1, wtp_b1, wtp_w2, wtp_b2,
        adj_row_block=400, ew_row_block=400,
        n_chunks=15)
    tt_edges, tt_weights = _edge_path(
        adj_tt, transition[0], target[0], ttp_w1, ttp_b1, ttp_w2, ttp_b2,
        adj_row_block=256, ew_row_block=256,
        n_chunks=2)
    return (wt_edges, wt_weights, tt_edges, tt_weights)
